# baseline (speedup 1.0000x reference)
import functools

import jax
import jax.numpy as jnp
from jax import lax
from jax.experimental import pallas as pl
from jax.experimental.pallas import tpu as pltpu
from jax.experimental.pallas import tpu_sc as plsc

NUM_EMBEDDINGS = 1000000
D = 64
BATCH = 4096
SEQ_LEN = 200
B = BATCH * SEQ_LEN

NC = 2
NS = 16
NW = NC * NS
BPW = B // NW

CH = 128
GPG = 8
GROUP = CH * GPG
NGROUPS = BPW // GROUP


def _make_kernel():
    mesh = plsc.VectorSubcoreMesh(core_axis_name="c", subcore_axis_name="s")

    @functools.partial(
        pl.kernel,
        out_type=jax.ShapeDtypeStruct((B, D), jnp.float32),
        mesh=mesh,
        scratch_types=[
            pltpu.VMEM((GROUP,), jnp.int32),
            pltpu.VMEM((GROUP, D), jnp.float32),
            pltpu.SemaphoreType.DMA,
        ],
        compiler_params=pltpu.CompilerParams(use_tc_tiling_on_sc=False),
    )
    def emb_kernel(idx_hbm, table_hbm, out_hbm, idx_v, rows_v, sem):
        wid = lax.axis_index("s") * NC + lax.axis_index("c")
        base = wid * BPW

        def body(g, carry):
            row0 = base + g * GROUP
            pltpu.sync_copy(idx_hbm.at[pl.ds(row0, GROUP)], idx_v)
            copies = []
            for j in range(GPG):
                copies.append(
                    pltpu.async_copy(
                        table_hbm.at[idx_v.at[pl.ds(j * CH, CH)]],
                        rows_v.at[pl.ds(j * CH, CH)],
                        sem,
                    )
                )
            for c in copies:
                c.wait()
            pltpu.sync_copy(rows_v, out_hbm.at[pl.ds(row0, GROUP)])
            return carry

        lax.fori_loop(0, NGROUPS, body, None)

    return emb_kernel


_emb_kernel = _make_kernel()


def kernel(input_, W):
    idx = input_.reshape(B).astype(jnp.int32)
    out = _emb_kernel(idx, W)
    return out.reshape(BATCH, SEQ_LEN, D)
